# Initial kernel scaffold; baseline (speedup 1.0000x reference)
import functools

import jax
import jax.numpy as jnp
from jax import lax
from jax.experimental import pallas as pl
from jax.experimental.pallas import tpu as pltpu
from jax.experimental.pallas import tpu_sc as plsc

B, L, C = 64, 2048, 8
K, V, ED = 26, 100000, 16
CONT = 8
F = 1 + CONT + K * ED
EPS = 1e-5
LN = 16
NC = 2
NW = 32
BPW = B // NW
CH = 128
NCH = L // CH


def _rsqrt16(v):
    bits = lax.bitcast_convert_type(v, jnp.int32)
    i = jnp.int32(0x5F3759DF) - lax.shift_right_logical(bits, 1)
    y = lax.bitcast_convert_type(i, jnp.float32)
    for _ in range(3):
        y = y * (1.5 - 0.5 * v * y * y)
    return y


_mesh = plsc.VectorSubcoreMesh(core_axis_name="c", subcore_axis_name="s")


@functools.partial(
    pl.kernel,
    out_type=jax.ShapeDtypeStruct((B * L, F), jnp.float32),
    mesh=_mesh,
    scratch_types=[
        pltpu.VMEM((L * C,), jnp.float32),
        pltpu.VMEM((L * CONT,), jnp.float32),
        pltpu.VMEM((CH * K,), jnp.int32),
        pltpu.VMEM((K, CH), jnp.int32),
        pltpu.VMEM((K, CH, ED), jnp.float32),
        pltpu.VMEM((CH, 1 + CONT), jnp.float32),
        pltpu.SemaphoreType.DMA,
    ],
)
def _titan_sc(xf, cf, catf, rw, rb, tab, out,
              xb, cb, ixr, ixt, emb, head, sem):
    wid = lax.axis_index("s") * NC + lax.axis_index("c")
    iota = lax.iota(jnp.int32, LN)

    def batch_body(bi, _):
        b = wid * BPW + bi
        pltpu.sync_copy(xf.at[b], xb)
        pltpu.sync_copy(cf.at[b], cb)

        s = jnp.zeros((LN,), jnp.float32)
        s2 = jnp.zeros((LN,), jnp.float32)
        for v in range(L // LN):
            xv = plsc.load_gather(xb, [(iota + v * LN) * C])
            s = s + xv
            s2 = s2 + xv * xv
        inv_n = jnp.float32(1.0 / L)
        mu = jnp.sum(s) * inv_n
        var = jnp.sum(s2) * inv_n - mu * mu
        inv = _rsqrt16(jnp.full((LN,), var + EPS, jnp.float32))
        a = inv * rw[...]
        c0 = rb[...] - mu * a

        def chunk_body(ci, _c):
            l0 = ci * CH
            row0 = b * L + l0
            pltpu.sync_copy(catf.at[b, pl.ds(l0 * K, CH * K)], ixr)
            for i in range(K):
                ofs = jnp.full((LN,), i * V, jnp.int32)
                for v in range(CH // LN):
                    g = plsc.load_gather(ixr, [(iota + v * LN) * K + i])
                    ixt[i, pl.ds(v * LN, LN)] = g + ofs
            cps = []
            for i in range(K):
                cps.append(pltpu.async_copy(tab.at[ixt.at[i]], emb.at[i], sem))
            z = jnp.zeros((LN,), jnp.int32)
            for v in range(CH // LN):
                p16 = iota + v * LN
                xv = plsc.load_gather(xb, [(l0 + p16) * C])
                plsc.store_scatter(head, [p16, z], xv * a + c0)
            for v in range(CH * CONT // LN):
                cix = iota + v * LN
                p = lax.shift_right_logical(cix, 3)
                q = cix - p * CONT
                cv = cb[pl.ds(l0 * CONT + v * LN, LN)]
                plsc.store_scatter(head, [p, q + 1], cv)
            pltpu.sync_copy(head, out.at[pl.ds(row0, CH), pl.ds(0, 1 + CONT)])
            for i in range(K):
                cps[i].wait()
                pltpu.sync_copy(
                    emb.at[i],
                    out.at[pl.ds(row0, CH), pl.ds(1 + CONT + i * ED, ED)])
            return _c

        return lax.fori_loop(0, NCH, chunk_body, _)

    lax.fori_loop(0, BPW, batch_body, None)


def kernel(x, past_exo_cont, past_exo_cat, revin_weight, revin_bias,
           emb_tables):
    xf = x.reshape(B, L * C)
    cf = past_exo_cont.reshape(B, L * CONT)
    catf = past_exo_cat.astype(jnp.int32).reshape(B, L * K)
    tab = emb_tables.reshape(K * V, ED)
    rw = jnp.broadcast_to(revin_weight.astype(jnp.float32), (LN,))
    rb = jnp.broadcast_to(revin_bias.astype(jnp.float32), (LN,))
    out = _titan_sc(xf, cf, catf, rw, rb, tab)
    return out.reshape(B, L, F)


# speedup vs baseline: 1.1511x; 1.1511x over previous
import functools

import jax
import jax.numpy as jnp
from jax import lax
from jax.experimental import pallas as pl
from jax.experimental.pallas import tpu as pltpu
from jax.experimental.pallas import tpu_sc as plsc

B, L, C = 64, 2048, 8
K, V, ED = 26, 100000, 16
CONT = 8
F = 1 + CONT + K * ED
EPS = 1e-5
LN = 16
NC = 2
NW = 32
BPW = B // NW
CH = 128
NCH = L // CH


def _rsqrt16(v):
    bits = lax.bitcast_convert_type(v, jnp.int32)
    i = jnp.int32(0x5F3759DF) - lax.shift_right_logical(bits, 1)
    y = lax.bitcast_convert_type(i, jnp.float32)
    for _ in range(3):
        y = y * (1.5 - 0.5 * v * y * y)
    return y


_mesh = plsc.VectorSubcoreMesh(core_axis_name="c", subcore_axis_name="s")


@functools.partial(
    pl.kernel,
    out_type=jax.ShapeDtypeStruct((B * L, 7 + F), jnp.float32),
    mesh=_mesh,
    compiler_params=pltpu.CompilerParams(
        use_tc_tiling_on_sc=False, needs_layout_passes=False),
    scratch_types=[
        pltpu.VMEM((L * C,), jnp.float32),
        pltpu.VMEM((L * CONT,), jnp.float32),
        pltpu.VMEM((CH * K,), jnp.int32),
        pltpu.VMEM((K, CH), jnp.int32),
        pltpu.VMEM((K, CH, ED), jnp.float32),
        pltpu.VMEM((CH, 16), jnp.float32),
        pltpu.VMEM((LN,), jnp.float32),
        pltpu.VMEM((LN,), jnp.float32),
        pltpu.SemaphoreType.DMA,
    ],
)
def _titan_sc(xf, cf, catf, rw, rb, tab, out,
              xb, cb, ixr, ixt, emb, head, rwv, rbv, sem):
    wid = lax.axis_index("s") * NC + lax.axis_index("c")
    iota = lax.iota(jnp.int32, LN)
    pltpu.sync_copy(rw, rwv)
    pltpu.sync_copy(rb, rbv)
    w16 = rwv[...]
    b16 = rbv[...]

    def batch_body(bi, _):
        b = wid * BPW + bi
        pltpu.sync_copy(xf.at[b], xb)
        pltpu.sync_copy(cf.at[b], cb)

        s = jnp.zeros((LN,), jnp.float32)
        s2 = jnp.zeros((LN,), jnp.float32)
        for v in range(L // LN):
            xv = plsc.load_gather(xb, [(iota + v * LN) * C])
            s = s + xv
            s2 = s2 + xv * xv
        inv_n = jnp.float32(1.0 / L)
        mu = jnp.sum(s) * inv_n
        var = jnp.sum(s2) * inv_n - mu * mu
        inv = _rsqrt16(jnp.full((LN,), var + EPS, jnp.float32))
        a = inv * w16
        c0 = b16 - mu * a

        def chunk_body(ci, _c):
            l0 = ci * CH
            row0 = b * L + l0
            pltpu.sync_copy(catf.at[b, pl.ds(l0 * K, CH * K)], ixr)
            for i in range(K):
                ofs = jnp.full((LN,), i * V, jnp.int32)
                for v in range(CH // LN):
                    g = plsc.load_gather(ixr, [(iota + v * LN) * K + i])
                    ixt[i, pl.ds(v * LN, LN)] = g + ofs
            cps = []
            for i in range(K):
                cps.append(pltpu.async_copy(tab.at[ixt.at[i]], emb.at[i], sem))
            s7 = jnp.full((LN,), 7, jnp.int32)
            for v in range(CH // LN):
                p16 = iota + v * LN
                xv = plsc.load_gather(xb, [(l0 + p16) * C])
                plsc.store_scatter(head, [p16, s7], xv * a + c0)
            for v in range(CH * CONT // LN):
                cix = iota + v * LN
                p = lax.shift_right_logical(cix, 3)
                q = cix - p * CONT
                cv = cb[pl.ds(l0 * CONT + v * LN, LN)]
                plsc.store_scatter(head, [p, q + 8], cv)
            pltpu.sync_copy(head, out.at[pl.ds(row0, CH), pl.ds(0, 16)])
            for i in range(K):
                cps[i].wait()
                pltpu.sync_copy(
                    emb.at[i],
                    out.at[pl.ds(row0, CH), pl.ds(16 + i * ED, ED)])
            return _c

        return lax.fori_loop(0, NCH, chunk_body, _)

    lax.fori_loop(0, BPW, batch_body, None)


def kernel(x, past_exo_cont, past_exo_cat, revin_weight, revin_bias,
           emb_tables):
    xf = x.reshape(B, L * C)
    cf = past_exo_cont.reshape(B, L * CONT)
    catf = past_exo_cat.astype(jnp.int32).reshape(B, L * K)
    tab = emb_tables.reshape(K * V, ED)
    rw = jnp.broadcast_to(revin_weight.astype(jnp.float32), (LN,))
    rb = jnp.broadcast_to(revin_bias.astype(jnp.float32), (LN,))
    out = _titan_sc(xf, cf, catf, rw, rb, tab)
    return out[:, 7:].reshape(B, L, F)


# speedup vs baseline: 1.1753x; 1.0210x over previous
import functools

import jax
import jax.numpy as jnp
from jax import lax
from jax.experimental import pallas as pl
from jax.experimental.pallas import tpu as pltpu
from jax.experimental.pallas import tpu_sc as plsc

B, L, C = 64, 2048, 8
K, V, ED = 26, 100000, 16
CONT = 8
F = 1 + CONT + K * ED
EPS = 1e-5
LN = 16
NC = 2
NW = 32
BPW = B // NW
CH = 128
NCH = L // CH
LT = L // 128


def _rsqrt16(v):
    bits = lax.bitcast_convert_type(v, jnp.int32)
    i = jnp.int32(0x5F3759DF) - lax.shift_right_logical(bits, 1)
    y = lax.bitcast_convert_type(i, jnp.float32)
    for _ in range(3):
        y = y * (1.5 - 0.5 * v * y * y)
    return y


_mesh = plsc.VectorSubcoreMesh(core_axis_name="c", subcore_axis_name="s")


@functools.partial(
    pl.kernel,
    out_type=jax.ShapeDtypeStruct((B * L, 7 + F), jnp.float32),
    mesh=_mesh,
    compiler_params=pltpu.CompilerParams(
        use_tc_tiling_on_sc=False, needs_layout_passes=False),
    scratch_types=[
        pltpu.VMEM((L,), jnp.float32),
        pltpu.VMEM((CONT, CH), jnp.float32),
        pltpu.VMEM((K, CH), jnp.int32),
        pltpu.VMEM((K, CH, ED), jnp.float32),
        pltpu.VMEM((CH, 16), jnp.float32),
        pltpu.VMEM((LN,), jnp.float32),
        pltpu.VMEM((LN,), jnp.float32),
        pltpu.SemaphoreType.DMA,
    ],
)
def _titan_sc(xp, cp, catp, rw, rb, tab, out,
              xb, cb, ixt, emb, head, rwv, rbv, sem):
    wid = lax.axis_index("s") * NC + lax.axis_index("c")
    iota = lax.iota(jnp.int32, LN)
    pltpu.sync_copy(rw, rwv)
    pltpu.sync_copy(rb, rbv)
    w16 = rwv[...]
    b16 = rbv[...]

    def batch_body(bi, _):
        b = wid * BPW + bi
        b8 = b // 8
        br = b - b8 * 8
        def xload(lt, _x):
            pltpu.sync_copy(xp.at[b, lt, 0], xb.at[pl.ds(lt * 128, 128)])
            return _x
        lax.fori_loop(0, LT, xload, None)

        s = jnp.zeros((LN,), jnp.float32)
        s2 = jnp.zeros((LN,), jnp.float32)
        for v in range(L // LN):
            xv = xb[pl.ds(v * LN, LN)]
            s = s + xv
            s2 = s2 + xv * xv
        inv_n = jnp.float32(1.0 / L)
        mu = jnp.sum(s) * inv_n
        var = jnp.sum(s2) * inv_n - mu * mu
        inv = _rsqrt16(jnp.full((LN,), var + EPS, jnp.float32))
        a = inv * w16
        c0 = b16 - mu * a

        def chunk_body(ci, _c):
            l0 = ci * CH
            row0 = b * L + l0
            pltpu.sync_copy(catp.at[:, b8, ci, br], ixt)
            for i in range(K):
                ofs = jnp.full((LN,), i * V, jnp.int32)
                for v in range(CH // LN):
                    sl = pl.ds(v * LN, LN)
                    ixt[i, sl] = ixt[i, sl] + ofs
            cps = []
            for i in range(K):
                cps.append(pltpu.async_copy(tab.at[ixt.at[i]], emb.at[i], sem))
            pltpu.sync_copy(cp.at[b, ci], cb)
            s7 = jnp.full((LN,), 7, jnp.int32)
            for v in range(CH // LN):
                p16 = iota + v * LN
                xv = xb[pl.ds(l0 + v * LN, LN)]
                plsc.store_scatter(head, [p16, s7], xv * a + c0)
            for q in range(CONT):
                cq = jnp.full((LN,), q + 8, jnp.int32)
                for v in range(CH // LN):
                    cv = cb[q, pl.ds(v * LN, LN)]
                    plsc.store_scatter(head, [iota + v * LN, cq], cv)
            pltpu.sync_copy(head, out.at[pl.ds(row0, CH), pl.ds(0, 16)])
            for i in range(K):
                cps[i].wait()
                pltpu.sync_copy(
                    emb.at[i],
                    out.at[pl.ds(row0, CH), pl.ds(16 + i * ED, ED)])
            return _c

        return lax.fori_loop(0, NCH, chunk_body, _)

    lax.fori_loop(0, BPW, batch_body, None)


def kernel(x, past_exo_cont, past_exo_cat, revin_weight, revin_bias,
           emb_tables):
    xpv = x.transpose(0, 2, 1).reshape(B, C, LT, 128).transpose(0, 2, 1, 3)
    cpv = (past_exo_cont.transpose(0, 2, 1)
           .reshape(B, CONT, LT, 128).transpose(0, 2, 1, 3))
    catp = (past_exo_cat.astype(jnp.int32).transpose(2, 0, 1)
            .reshape(K, B // 8, 8, LT, 128).transpose(0, 1, 3, 2, 4))
    tab = emb_tables.reshape(K * V, ED)
    rw = jnp.broadcast_to(revin_weight.astype(jnp.float32), (LN,))
    rb = jnp.broadcast_to(revin_bias.astype(jnp.float32), (LN,))
    padded = _titan_sc(xpv, cpv, catp, rw, rb, tab)
    return padded[:, 7:].reshape(B, L, F)
